# Initial kernel scaffold; baseline (speedup 1.0000x reference)
import jax
import jax.numpy as jnp
from jax.experimental import pallas as pl


def kernel(history_items, next_items, history_corrects, item_embedding, item_beta_weights, item_response_vals, td_kernel, td_bias):
    raise NotImplementedError("write your pallas kernel here")


# speedup vs baseline: 1.0007x; 1.0007x over previous
import jax
import jax.numpy as jnp
from jax.experimental import pallas as pl
from jax.experimental.pallas import tpu as pltpu

NEG = -1e9


def _dpfa_attn_kernel(td_ref, next_ref, hist_ref, items_ref, mast_ref,
                      beta_ref, out_ref):
    S = next_ref.shape[1]
    nxt = next_ref[0]
    hist = hist_ref[0]
    nxt = nxt * jax.lax.rsqrt(jnp.sum(nxt * nxt, axis=1, keepdims=True))
    hist = hist * jax.lax.rsqrt(jnp.sum(hist * hist, axis=1, keepdims=True))
    scores = jax.lax.dot_general(nxt, hist, (((1,), (1,)), ((), ())),
                                 preferred_element_type=jnp.float32)
    q_iota = jax.lax.broadcasted_iota(jnp.int32, (S, S), 0)
    s_iota = jax.lax.broadcasted_iota(jnp.int32, (S, S), 1)
    causal = jnp.where(s_iota > q_iota, NEG, 0.0)
    pad_row = jnp.where(items_ref[0] == 0, NEG, 0.0)
    bias = jnp.minimum(pad_row, causal)
    dist = (q_iota + 1 - s_iota).astype(jnp.float32)
    tdm = td_ref[0] * dist + td_ref[1]
    logits = scores + bias + tdm
    m = jnp.max(logits, axis=1, keepdims=True)
    e = jnp.exp(logits - m)
    denom = jnp.sum(e, axis=1, keepdims=True)
    numer = jnp.sum(e * mast_ref[0], axis=1, keepdims=True)
    ability = numer / denom
    out_ref[0] = jax.nn.sigmoid(ability - beta_ref[0])


def kernel(history_items, next_items, history_corrects, item_embedding,
           item_beta_weights, item_response_vals, td_kernel, td_bias):
    B, S = history_items.shape
    H = item_embedding.shape[1]
    hist_emb = item_embedding[history_items]
    next_emb = item_embedding[next_items]
    is_correct = (history_corrects == 2).astype(jnp.int32)
    mastery = item_response_vals[history_items, is_correct]
    next_diff = item_beta_weights[next_items]

    td = jnp.concatenate([td_kernel, td_bias]).astype(jnp.float32)
    items_r = history_items.astype(jnp.int32).reshape(B, 1, S)
    mast_r = mastery.reshape(B, 1, S)
    beta_r = next_diff.reshape(B, S, 1)

    out = pl.pallas_call(
        _dpfa_attn_kernel,
        grid=(B,),
        in_specs=[
            pl.BlockSpec(memory_space=pltpu.SMEM),
            pl.BlockSpec((1, S, H), lambda b: (b, 0, 0)),
            pl.BlockSpec((1, S, H), lambda b: (b, 0, 0)),
            pl.BlockSpec((1, 1, S), lambda b: (b, 0, 0)),
            pl.BlockSpec((1, 1, S), lambda b: (b, 0, 0)),
            pl.BlockSpec((1, S, 1), lambda b: (b, 0, 0)),
        ],
        out_specs=pl.BlockSpec((1, S, 1), lambda b: (b, 0, 0)),
        out_shape=jax.ShapeDtypeStruct((B, S, 1), jnp.float32),
        compiler_params=pltpu.CompilerParams(
            dimension_semantics=("parallel",)),
    )(td, next_emb, hist_emb, items_r, mast_r, beta_r)
    return out.reshape(B, S)


# speedup vs baseline: 4.1525x; 4.1497x over previous
import jax
import jax.numpy as jnp
from jax.experimental import pallas as pl
from jax.experimental.pallas import tpu as pltpu

NEG = -1e9
H = 128
AUG = 256


def _norm_table_kernel(emb_ref, ext_ref, out_ref):
    x = emb_ref[...]
    ssq = jnp.sum(x * x, axis=1, keepdims=True)
    out_ref[:, 0:H] = x * jax.lax.rsqrt(ssq)
    out_ref[:, H:AUG] = ext_ref[...]


def _gather_rows(table_ref, idx_ref, slot_ref, n):
    for mi in range(n):
        idx = idx_ref[0, 0, mi]
        base = pl.multiple_of((idx >> 3) << 3, 8)
        chunk = table_ref[pl.ds(base, 8), :]
        row = pltpu.roll(chunk, -(idx & 7), axis=0)[0:1, :]
        slot_ref[mi:mi + 1, :] = row


def _dpfa_kernel(td_ref, table_ref, hist_sref, next_sref, items_ref,
                 corr_ref, out_ref, gh_ref, gn_ref):
    S = items_ref.shape[2]
    _gather_rows(table_ref, hist_sref, gh_ref, S)
    _gather_rows(table_ref, next_sref, gn_ref, S)

    hist_n = gh_ref[:, 0:H]
    nxt_n = gn_ref[:, 0:H]
    scores = jax.lax.dot_general(nxt_n, hist_n, (((1,), (1,)), ((), ())),
                                 preferred_element_type=jnp.float32)
    q_iota = jax.lax.broadcasted_iota(jnp.int32, (S, S), 0)
    s_iota = jax.lax.broadcasted_iota(jnp.int32, (S, S), 1)
    causal = jnp.where(s_iota > q_iota, NEG, 0.0)
    pad_row = jnp.where(items_ref[0] == 0, NEG, 0.0)
    bias = jnp.minimum(pad_row, causal)
    dist = (q_iota + 1 - s_iota).astype(jnp.float32)
    logits = scores + bias + td_ref[0] * dist + td_ref[1]
    m = jnp.max(logits, axis=1, keepdims=True)
    e = jnp.exp(logits - m)

    rv0 = gh_ref[:, 129:130]
    rv1 = gh_ref[:, 130:131]
    mast = jnp.where(corr_ref[0] == 2, rv1, rv0)
    w2 = jnp.concatenate([mast, jnp.ones_like(mast)], axis=1)
    nd = jnp.dot(e, w2, preferred_element_type=jnp.float32)
    ability = nd[:, 0:1] / nd[:, 1:2]
    beta = gn_ref[:, 128:129]
    out_ref[0] = jax.nn.sigmoid(ability - beta)


def kernel(history_items, next_items, history_corrects, item_embedding,
           item_beta_weights, item_response_vals, td_kernel, td_bias):
    B, S = history_items.shape
    V = item_embedding.shape[0]

    extras = jnp.concatenate(
        [item_beta_weights[:, None], item_response_vals,
         jnp.zeros((V, 125), dtype=jnp.float32)], axis=1)

    table = pl.pallas_call(
        _norm_table_kernel,
        grid=(10,),
        in_specs=[
            pl.BlockSpec((V // 10, H), lambda i: (i, 0)),
            pl.BlockSpec((V // 10, 128), lambda i: (i, 0)),
        ],
        out_specs=pl.BlockSpec((V // 10, AUG), lambda i: (i, 0)),
        out_shape=jax.ShapeDtypeStruct((V, AUG), jnp.float32),
        compiler_params=pltpu.CompilerParams(
            dimension_semantics=("parallel",)),
    )(item_embedding, extras)

    td = jnp.concatenate([td_kernel, td_bias]).astype(jnp.float32)
    hist_i = history_items.astype(jnp.int32).reshape(B, 1, S)
    next_i = next_items.astype(jnp.int32).reshape(B, 1, S)
    corr_c = history_corrects.astype(jnp.int32).reshape(B, S, 1)

    out = pl.pallas_call(
        _dpfa_kernel,
        grid=(B,),
        in_specs=[
            pl.BlockSpec(memory_space=pltpu.SMEM),
            pl.BlockSpec((V, AUG), lambda b: (0, 0)),
            pl.BlockSpec((1, 1, S), lambda b: (b, 0, 0),
                         memory_space=pltpu.SMEM),
            pl.BlockSpec((1, 1, S), lambda b: (b, 0, 0),
                         memory_space=pltpu.SMEM),
            pl.BlockSpec((1, 1, S), lambda b: (b, 0, 0)),
            pl.BlockSpec((1, S, 1), lambda b: (b, 0, 0)),
        ],
        out_specs=pl.BlockSpec((1, S, 1), lambda b: (b, 0, 0)),
        out_shape=jax.ShapeDtypeStruct((B, S, 1), jnp.float32),
        scratch_shapes=[
            pltpu.VMEM((S, AUG), jnp.float32),
            pltpu.VMEM((S, AUG), jnp.float32),
        ],
        compiler_params=pltpu.CompilerParams(
            dimension_semantics=("parallel",)),
    )(td, table, hist_i, next_i, hist_i, corr_c)
    return out.reshape(B, S)


# speedup vs baseline: 4.8492x; 1.1678x over previous
import jax
import jax.numpy as jnp
from jax.experimental import pallas as pl
from jax.experimental.pallas import tpu as pltpu

NEG = -1e9
H = 128
AUG = 256
G = 2


def _norm_table_kernel(emb_ref, ext_ref, out_ref):
    x = emb_ref[...]
    ssq = jnp.sum(x * x, axis=1, keepdims=True)
    out_ref[:, 0:H] = x * jax.lax.rsqrt(ssq)
    out_ref[:, H:AUG] = ext_ref[...]


def _gather_rows(table_ref, idx_ref, shift_ref, slot_ref, g, n):
    for mi in range(n):
        idx = idx_ref[g, 0, mi]
        base = pl.multiple_of((idx >> 3) << 3, 8)
        chunk = table_ref[pl.ds(base, 8), :]
        row = pltpu.roll(chunk, shift_ref[g, 0, mi], axis=0)[0:1, :]
        slot_ref[mi:mi + 1, :] = row


def _attention(td_ref, items_row, corr_col, gh, gn, out_ref):
    S = gh.shape[0]
    hist_n = gh[:, 0:H]
    nxt_n = gn[:, 0:H]
    scores = jax.lax.dot_general(nxt_n, hist_n, (((1,), (1,)), ((), ())),
                                 preferred_element_type=jnp.float32)
    q_iota = jax.lax.broadcasted_iota(jnp.int32, (S, S), 0)
    s_iota = jax.lax.broadcasted_iota(jnp.int32, (S, S), 1)
    causal = jnp.where(s_iota > q_iota, NEG, 0.0)
    pad_row = jnp.where(items_row == 0, NEG, 0.0)
    bias = jnp.minimum(pad_row, causal)
    dist = (q_iota + 1 - s_iota).astype(jnp.float32)
    logits = scores + bias + td_ref[0] * dist + td_ref[1]
    m = jnp.max(logits, axis=1, keepdims=True)
    e = jnp.exp(logits - m)

    rv0 = gh[:, 129:130]
    rv1 = gh[:, 130:131]
    mast = jnp.where(corr_col == 2, rv1, rv0)
    w2 = jnp.concatenate([mast, jnp.ones_like(mast)], axis=1)
    nd = jnp.dot(e, w2, preferred_element_type=jnp.float32)
    ability = nd[:, 0:1] / nd[:, 1:2]
    beta = gn[:, 128:129]
    out_ref[...] = jax.nn.sigmoid(ability - beta)


def _dpfa_kernel(td_ref, table_ref, hist_sref, hsh_ref, next_sref, nsh_ref,
                 items_ref, corr_ref, out_ref, gh_ref, gn_ref):
    S = items_ref.shape[2]
    for g in range(G):
        _gather_rows(table_ref, hist_sref, hsh_ref, gh_ref.at[g], g, S)
        _gather_rows(table_ref, next_sref, nsh_ref, gn_ref.at[g], g, S)
        _attention(td_ref, items_ref[g], corr_ref[g], gh_ref[g], gn_ref[g],
                   out_ref.at[g])


def kernel(history_items, next_items, history_corrects, item_embedding,
           item_beta_weights, item_response_vals, td_kernel, td_bias):
    B, S = history_items.shape
    V = item_embedding.shape[0]

    extras = jnp.concatenate(
        [item_beta_weights[:, None], item_response_vals,
         jnp.zeros((V, 125), dtype=jnp.float32)], axis=1)

    table = pl.pallas_call(
        _norm_table_kernel,
        grid=(10,),
        in_specs=[
            pl.BlockSpec((V // 10, H), lambda i: (i, 0)),
            pl.BlockSpec((V // 10, 128), lambda i: (i, 0)),
        ],
        out_specs=pl.BlockSpec((V // 10, AUG), lambda i: (i, 0)),
        out_shape=jax.ShapeDtypeStruct((V, AUG), jnp.float32),
        compiler_params=pltpu.CompilerParams(
            dimension_semantics=("parallel",)),
    )(item_embedding, extras)

    td = jnp.concatenate([td_kernel, td_bias]).astype(jnp.float32)
    hist_i = history_items.astype(jnp.int32).reshape(B, 1, S)
    next_i = next_items.astype(jnp.int32).reshape(B, 1, S)
    hist_sh = (8 - (hist_i & 7)) & 7
    next_sh = (8 - (next_i & 7)) & 7
    corr_c = history_corrects.astype(jnp.int32).reshape(B, S, 1)

    out = pl.pallas_call(
        _dpfa_kernel,
        grid=(B // G,),
        in_specs=[
            pl.BlockSpec(memory_space=pltpu.SMEM),
            pl.BlockSpec((V, AUG), lambda b: (0, 0)),
            pl.BlockSpec((G, 1, S), lambda b: (b, 0, 0),
                         memory_space=pltpu.SMEM),
            pl.BlockSpec((G, 1, S), lambda b: (b, 0, 0),
                         memory_space=pltpu.SMEM),
            pl.BlockSpec((G, 1, S), lambda b: (b, 0, 0),
                         memory_space=pltpu.SMEM),
            pl.BlockSpec((G, 1, S), lambda b: (b, 0, 0),
                         memory_space=pltpu.SMEM),
            pl.BlockSpec((G, 1, S), lambda b: (b, 0, 0)),
            pl.BlockSpec((G, S, 1), lambda b: (b, 0, 0)),
        ],
        out_specs=pl.BlockSpec((G, S, 1), lambda b: (b, 0, 0)),
        out_shape=jax.ShapeDtypeStruct((B, S, 1), jnp.float32),
        scratch_shapes=[
            pltpu.VMEM((G, S, AUG), jnp.float32),
            pltpu.VMEM((G, S, AUG), jnp.float32),
        ],
        compiler_params=pltpu.CompilerParams(
            dimension_semantics=("parallel",)),
    )(td, table, hist_i, hist_sh, next_i, next_sh, hist_i, corr_c)
    return out.reshape(B, S)


# speedup vs baseline: 6.0923x; 1.2563x over previous
import jax
import jax.numpy as jnp
from jax.experimental import pallas as pl
from jax.experimental.pallas import tpu as pltpu

NEG = -1e9
H = 128
G = 2


def _norm_table_kernel(emb_ref, out_ref):
    x = emb_ref[...]
    ssq = jnp.sum(x * x, axis=1, keepdims=True)
    out_ref[...] = (x * jax.lax.rsqrt(ssq)).reshape(x.shape[0], 1, H)


def _gather_rows(emb_t, ext_t, idx_ref, emb_slot, ext_slot, g, base, n):
    for mi in range(n):
        idx = idx_ref[g, 0, mi]
        emb_slot[pl.ds(base + mi, 1)] = emb_t[pl.ds(idx, 1)]
        ext_slot[pl.ds(base + mi, 1)] = ext_t[pl.ds(idx, 1)]


def _attention(td_ref, items_row, corr_col, gh, gn, hx, nx, out_ref):
    S = gh.shape[0]
    scores = jax.lax.dot_general(gn, gh, (((1,), (1,)), ((), ())),
                                 preferred_element_type=jnp.float32)
    q_iota = jax.lax.broadcasted_iota(jnp.int32, (S, S), 0)
    s_iota = jax.lax.broadcasted_iota(jnp.int32, (S, S), 1)
    causal = jnp.where(s_iota > q_iota, NEG, 0.0)
    pad_row = jnp.where(items_row == 0, NEG, 0.0)
    bias = jnp.minimum(pad_row, causal)
    dist = (q_iota + 1 - s_iota).astype(jnp.float32)
    logits = scores + bias + td_ref[0] * dist + td_ref[1]
    m = jnp.max(logits, axis=1, keepdims=True)
    e = jnp.exp(logits - m)

    rv0 = hx[:, 1:2]
    rv1 = hx[:, 2:3]
    mast = jnp.where(corr_col == 2, rv1, rv0)
    w2 = jnp.concatenate([mast, jnp.ones_like(mast)], axis=1)
    nd = jnp.dot(e, w2, preferred_element_type=jnp.float32)
    ability = nd[:, 0:1] / nd[:, 1:2]
    beta = nx[:, 0:1]
    out_ref[...] = jax.nn.sigmoid(ability - beta)


def _dpfa_kernel(td_ref, emb_t, ext_t, hist_sref, next_sref,
                 items_ref, corr_ref, out_ref, he_ref, hx_ref,
                 ne_ref, nx_ref):
    S = items_ref.shape[2]
    for g in range(G):
        _gather_rows(emb_t, ext_t, hist_sref, he_ref, hx_ref, g, g * S, S)
        _gather_rows(emb_t, ext_t, next_sref, ne_ref, nx_ref, g, g * S, S)
        gh = he_ref[g * S:(g + 1) * S].reshape(S, H)
        gn = ne_ref[g * S:(g + 1) * S].reshape(S, H)
        hx = hx_ref[g * S:(g + 1) * S].reshape(S, 128)
        nx = nx_ref[g * S:(g + 1) * S].reshape(S, 128)
        _attention(td_ref, items_ref[g], corr_ref[g], gh, gn, hx, nx,
                   out_ref.at[g])


def kernel(history_items, next_items, history_corrects, item_embedding,
           item_beta_weights, item_response_vals, td_kernel, td_bias):
    B, S = history_items.shape
    V = item_embedding.shape[0]

    extras = jnp.concatenate(
        [item_beta_weights[:, None], item_response_vals,
         jnp.zeros((V, 125), dtype=jnp.float32)], axis=1).reshape(V, 1, 128)

    emb_t = pl.pallas_call(
        _norm_table_kernel,
        grid=(10,),
        in_specs=[pl.BlockSpec((V // 10, H), lambda i: (i, 0))],
        out_specs=pl.BlockSpec((V // 10, 1, H), lambda i: (i, 0, 0)),
        out_shape=jax.ShapeDtypeStruct((V, 1, H), jnp.float32),
        compiler_params=pltpu.CompilerParams(
            dimension_semantics=("parallel",)),
    )(item_embedding)

    td = jnp.concatenate([td_kernel, td_bias]).astype(jnp.float32)
    hist_i = history_items.astype(jnp.int32).reshape(B, 1, S)
    next_i = next_items.astype(jnp.int32).reshape(B, 1, S)
    corr_c = history_corrects.astype(jnp.int32).reshape(B, S, 1)

    out = pl.pallas_call(
        _dpfa_kernel,
        grid=(B // G,),
        in_specs=[
            pl.BlockSpec(memory_space=pltpu.SMEM),
            pl.BlockSpec((V, 1, H), lambda b: (0, 0, 0)),
            pl.BlockSpec((V, 1, 128), lambda b: (0, 0, 0)),
            pl.BlockSpec((G, 1, S), lambda b: (b, 0, 0),
                         memory_space=pltpu.SMEM),
            pl.BlockSpec((G, 1, S), lambda b: (b, 0, 0),
                         memory_space=pltpu.SMEM),
            pl.BlockSpec((G, 1, S), lambda b: (b, 0, 0)),
            pl.BlockSpec((G, S, 1), lambda b: (b, 0, 0)),
        ],
        out_specs=pl.BlockSpec((G, S, 1), lambda b: (b, 0, 0)),
        out_shape=jax.ShapeDtypeStruct((B, S, 1), jnp.float32),
        scratch_shapes=[
            pltpu.VMEM((G * S, 1, H), jnp.float32),
            pltpu.VMEM((G * S, 1, 128), jnp.float32),
            pltpu.VMEM((G * S, 1, H), jnp.float32),
            pltpu.VMEM((G * S, 1, 128), jnp.float32),
        ],
        compiler_params=pltpu.CompilerParams(
            dimension_semantics=("parallel",)),
    )(td, emb_t, extras, hist_i, next_i, hist_i, corr_c)
    return out.reshape(B, S)


# speedup vs baseline: 6.2359x; 1.0236x over previous
import jax
import jax.numpy as jnp
from jax.experimental import pallas as pl
from jax.experimental.pallas import tpu as pltpu

NEG = -1e9
H = 128
G = 4


def _norm_table_kernel(emb_ref, out_ref):
    x = emb_ref[...]
    ssq = jnp.sum(x * x, axis=1, keepdims=True)
    out_ref[...] = (x * jax.lax.rsqrt(ssq)).reshape(x.shape[0], 1, H)


def _gather_rows(emb_t, ext_t, idx_ref, emb_slot, ext_slot, g, base, n):
    for mi in range(n):
        idx = idx_ref[g, 0, mi]
        emb_slot[pl.ds(base + mi, 1)] = emb_t[pl.ds(idx, 1)]
        ext_slot[pl.ds(base + mi, 1)] = ext_t[pl.ds(idx, 1)]


def _attention(td_ref, items_row, corr_col, gh, gn, hx, nx, out_ref):
    S = gh.shape[0]
    scores = jax.lax.dot_general(gn, gh, (((1,), (1,)), ((), ())),
                                 preferred_element_type=jnp.float32)
    q_iota = jax.lax.broadcasted_iota(jnp.int32, (S, S), 0)
    s_iota = jax.lax.broadcasted_iota(jnp.int32, (S, S), 1)
    causal = jnp.where(s_iota > q_iota, NEG, 0.0)
    pad_row = jnp.where(items_row == 0, NEG, 0.0)
    bias = jnp.minimum(pad_row, causal)
    dist = (q_iota + 1 - s_iota).astype(jnp.float32)
    logits = scores + bias + td_ref[0] * dist + td_ref[1]
    m = jnp.max(logits, axis=1, keepdims=True)
    e = jnp.exp(logits - m)

    rv0 = hx[:, 1:2]
    rv1 = hx[:, 2:3]
    mast = jnp.where(corr_col == 2, rv1, rv0)
    w2 = jnp.concatenate([mast, jnp.ones_like(mast)], axis=1)
    nd = jnp.dot(e, w2, preferred_element_type=jnp.float32)
    ability = nd[:, 0:1] / nd[:, 1:2]
    beta = nx[:, 0:1]
    out_ref[...] = jax.nn.sigmoid(ability - beta)


def _dpfa_kernel(td_ref, emb_t, ext_t, hist_sref, next_sref,
                 items_ref, corr_ref, out_ref, he_ref, hx_ref,
                 ne_ref, nx_ref):
    S = items_ref.shape[2]
    for g in range(G):
        _gather_rows(emb_t, ext_t, hist_sref, he_ref, hx_ref, g, g * S, S)
        _gather_rows(emb_t, ext_t, next_sref, ne_ref, nx_ref, g, g * S, S)
        gh = he_ref[g * S:(g + 1) * S].reshape(S, H)
        gn = ne_ref[g * S:(g + 1) * S].reshape(S, H)
        hx = hx_ref[g * S:(g + 1) * S].reshape(S, 128)
        nx = nx_ref[g * S:(g + 1) * S].reshape(S, 128)
        _attention(td_ref, items_ref[g], corr_ref[g], gh, gn, hx, nx,
                   out_ref.at[g])


def kernel(history_items, next_items, history_corrects, item_embedding,
           item_beta_weights, item_response_vals, td_kernel, td_bias):
    B, S = history_items.shape
    V = item_embedding.shape[0]

    extras = jnp.concatenate(
        [item_beta_weights[:, None], item_response_vals,
         jnp.zeros((V, 125), dtype=jnp.float32)], axis=1).reshape(V, 1, 128)

    emb_t = pl.pallas_call(
        _norm_table_kernel,
        grid=(10,),
        in_specs=[pl.BlockSpec((V // 10, H), lambda i: (i, 0))],
        out_specs=pl.BlockSpec((V // 10, 1, H), lambda i: (i, 0, 0)),
        out_shape=jax.ShapeDtypeStruct((V, 1, H), jnp.float32),
        compiler_params=pltpu.CompilerParams(
            dimension_semantics=("parallel",)),
    )(item_embedding)

    td = jnp.concatenate([td_kernel, td_bias]).astype(jnp.float32)
    hist_i = history_items.astype(jnp.int32).reshape(B, 1, S)
    next_i = next_items.astype(jnp.int32).reshape(B, 1, S)
    corr_c = history_corrects.astype(jnp.int32).reshape(B, S, 1)

    out = pl.pallas_call(
        _dpfa_kernel,
        grid=(B // G,),
        in_specs=[
            pl.BlockSpec(memory_space=pltpu.SMEM),
            pl.BlockSpec((V, 1, H), lambda b: (0, 0, 0)),
            pl.BlockSpec((V, 1, 128), lambda b: (0, 0, 0)),
            pl.BlockSpec((G, 1, S), lambda b: (b, 0, 0),
                         memory_space=pltpu.SMEM),
            pl.BlockSpec((G, 1, S), lambda b: (b, 0, 0),
                         memory_space=pltpu.SMEM),
            pl.BlockSpec((G, 1, S), lambda b: (b, 0, 0)),
            pl.BlockSpec((G, S, 1), lambda b: (b, 0, 0)),
        ],
        out_specs=pl.BlockSpec((G, S, 1), lambda b: (b, 0, 0)),
        out_shape=jax.ShapeDtypeStruct((B, S, 1), jnp.float32),
        scratch_shapes=[
            pltpu.VMEM((G * S, 1, H), jnp.float32),
            pltpu.VMEM((G * S, 1, 128), jnp.float32),
            pltpu.VMEM((G * S, 1, H), jnp.float32),
            pltpu.VMEM((G * S, 1, 128), jnp.float32),
        ],
        compiler_params=pltpu.CompilerParams(
            dimension_semantics=("parallel",)),
    )(td, emb_t, extras, hist_i, next_i, hist_i, corr_c)
    return out.reshape(B, S)
